# baseline (speedup 1.0000x reference)
import functools
import math

import jax
import jax.numpy as jnp
from jax import lax
from jax.experimental import pallas as pl
from jax.experimental.pallas import tpu as pltpu

MAX_N = 3
MAX_L = 3
CUTOFF = 5.0
THREE_CUTOFF = 4.0
NNF = 16
NEF = 16
UNITS = 64
N_BLOCKS = 3
N_ELEM = 16
DEGREE = MAX_N * MAX_L

TE = 2000
TT = 2000
TN = 2000


def _sigmoid(x):
    return 1.0 / (1.0 + jnp.exp(-x))


def _swish(x):
    return x * _sigmoid(x)


def _smooth_sbf_cols(bd):
    pi = math.pi
    cols = []
    for n in range(MAX_N):
        x1 = bd * ((n + 1.0) * pi / CUTOFF)
        x2 = bd * ((n + 2.0) * pi / CUTOFF)
        c = ((-1.0) ** n) * math.sqrt(2.0) * pi / CUTOFF ** 1.5 \
            * (n + 1.0) * (n + 2.0) / math.sqrt((n + 1.0) ** 2 + (n + 2.0) ** 2)
        cols.append(c * (jnp.sin(x1) / x1 + jnp.sin(x2) / x2))
    dn = [1.0]
    for i in range(1, MAX_N):
        e_i = i ** 2 * (i + 2) ** 2 / (4.0 * (i + 1) ** 4 + 1.0)
        dn.append(1.0 - e_i / dn[-1])
    gn = [cols[0]]
    for i in range(1, MAX_N):
        e_i = i ** 2 * (i + 2) ** 2 / (4.0 * (i + 1) ** 4 + 1.0)
        gn.append((cols[i] + math.sqrt(e_i / dn[i - 1]) * gn[-1]) / math.sqrt(dn[i]))
    return jnp.concatenate(gn, axis=1)


def _poly_cutoff(r, rc):
    ratio = r / rc
    v = 1.0 - 6.0 * ratio ** 5 + 15.0 * ratio ** 4 - 10.0 * ratio ** 3
    return jnp.where(r <= rc, v, 0.0)


def _geom_body(psrc, pdst, wee, bee, wew, wnw, g_out, ef0, ew, nw):
    bv = pdst[:, 0:3] - psrc[:, 0:3]
    bd = jnp.sqrt(jnp.sum(bv * bv, axis=1, keepdims=True) + 1e-12)
    cut = _poly_cutoff(bd, THREE_CUTOFF)
    rbf = _smooth_sbf_cols(bd)
    zcols = jnp.zeros((psrc.shape[0], 11), jnp.float32)
    g_out[...] = jnp.concatenate([bv, bd, cut, zcols], axis=1)
    ef0[...] = _swish(jnp.dot(rbf, wee[...], preferred_element_type=jnp.float32)
                      + bee[...])
    ew[...] = jnp.dot(rbf, wew[...], preferred_element_type=jnp.float32)
    nw[...] = jnp.dot(rbf, wnw[...], preferred_element_type=jnp.float32)


def _geom_call(psrc, pdst, wee, bee, wew, wnw):
    e = psrc.shape[0]
    grid = e // TE
    eb = lambda w: pl.BlockSpec((TE, w), lambda i: (i, 0))
    wb = lambda a: pl.BlockSpec(a.shape, lambda i: (0,) * a.ndim)
    return pl.pallas_call(
        _geom_body,
        grid=(grid,),
        in_specs=[eb(4), eb(4), wb(wee), wb(bee), wb(wew), wb(wnw)],
        out_specs=[eb(16), eb(16), eb(16), eb(16)],
        out_shape=[jax.ShapeDtypeStruct((e, 16), jnp.float32)] * 4,
        compiler_params=pltpu.CompilerParams(
            dimension_semantics=("parallel",)),
    )(psrc, pdst, wee, bee, wew, wnw)


def _triple_body(ga, gb, basisw):
    va = ga[:, 0:3]
    vb = gb[:, 0:3]
    na = ga[:, 3:4]
    nb = gb[:, 3:4]
    cos_t = jnp.clip(jnp.sum(va * vb, axis=1, keepdims=True) / (na * nb),
                     -1.0, 1.0)
    sbf = _smooth_sbf_cols(nb)
    p0 = jnp.ones_like(cos_t)
    p1 = cos_t
    p2 = (3.0 * cos_t * cos_t - 1.0) * 0.5
    leg = jnp.concatenate([p0, p1, p2], axis=1)
    w = ga[:, 4:5] * gb[:, 4:5]
    outer = [sbf[:, i:i + 1] * leg for i in range(MAX_N)]
    z = jnp.zeros((ga.shape[0], 16 - DEGREE), jnp.float32)
    basisw[...] = jnp.concatenate(outer + [z], axis=1) * w


def _triple_call(ga, gb):
    t = ga.shape[0]
    grid = t // TT
    tb = lambda w: pl.BlockSpec((TT, w), lambda i: (i, 0))
    return pl.pallas_call(
        _triple_body,
        grid=(grid,),
        in_specs=[tb(16), tb(16)],
        out_specs=tb(16),
        out_shape=jax.ShapeDtypeStruct((t, 16), jnp.float32),
        compiler_params=pltpu.CompilerParams(
            dimension_semantics=("parallel",)),
    )(ga, gb)


def _prep0_body(nt, wemb, bemb, watom, batom, table):
    n = nt.shape[0]
    oh = (nt[...] == lax.broadcasted_iota(jnp.int32, (n, N_ELEM), 1)).astype(jnp.float32)
    nf = _swish(jnp.dot(oh, wemb[...], preferred_element_type=jnp.float32)
                + bemb[...])
    atoms = _sigmoid(jnp.dot(nf, watom[...], preferred_element_type=jnp.float32)
                     + batom[...])
    z = jnp.zeros((n, 32 - NNF - DEGREE), jnp.float32)
    table[...] = jnp.concatenate([nf, atoms, z], axis=1)


def _prep0_call(node_type, wemb, bemb, watom, batom):
    n = node_type.shape[0]
    grid = n // TN
    wb = lambda a: pl.BlockSpec(a.shape, lambda i: (0,) * a.ndim)
    return pl.pallas_call(
        _prep0_body,
        grid=(grid,),
        in_specs=[pl.BlockSpec((TN, 1), lambda i: (i, 0)),
                  wb(wemb), wb(bemb), wb(watom), wb(batom)],
        out_specs=pl.BlockSpec((TN, 32), lambda i: (i, 0)),
        out_shape=jax.ShapeDtypeStruct((n, 32), jnp.float32),
        compiler_params=pltpu.CompilerParams(
            dimension_semantics=("parallel",)),
    )(node_type, wemb, bemb, watom, batom)


def _prep_body(tprev, p0, p1, watom, batom, table):
    n = tprev.shape[0]
    nf = tprev[:, 0:NNF] + p0[...] + p1[...]
    atoms = _sigmoid(jnp.dot(nf, watom[...], preferred_element_type=jnp.float32)
                     + batom[...])
    z = jnp.zeros((n, 32 - NNF - DEGREE), jnp.float32)
    table[...] = jnp.concatenate([nf, atoms, z], axis=1)


def _prep_call(tprev, p0, p1, watom, batom):
    n = tprev.shape[0]
    grid = n // TN
    wb = lambda a: pl.BlockSpec(a.shape, lambda i: (0,) * a.ndim)
    nb = lambda w: pl.BlockSpec((TN, w), lambda i: (i, 0))
    return pl.pallas_call(
        _prep_body,
        grid=(grid,),
        in_specs=[nb(32), nb(16), nb(16), wb(watom), wb(batom)],
        out_specs=nb(32),
        out_shape=jax.ShapeDtypeStruct((n, 32), jnp.float32),
        compiler_params=pltpu.CompilerParams(
            dimension_semantics=("parallel",)),
    )(tprev, p0, p1, watom, batom)


def _conv_body(gsrc, gdst, s, ef, ew, nw, wbp, wbg,
               ep1, ep2, ep3, eg1, eg2, eg3, np1, np2, np3, ng1, ng2, ng3,
               ef_out, msg_out):
    f32 = jnp.float32
    dot = lambda a, b: jnp.dot(a, b, preferred_element_type=f32)
    src16 = gsrc[:, 0:16]
    dst16 = gdst[:, 0:16]
    atoms = gdst[:, 16:32]
    nb = s[...] * atoms
    bond = _swish(dot(nb, wbp[...])) * _sigmoid(dot(nb, wbg[...]))
    e1 = ef[...] + bond

    def sd(w1):
        return dot(src16, w1[0:16, :]) + dot(dst16, w1[16:32, :])

    sd_ep, sd_eg = sd(ep1[...]), sd(eg1[...])
    hp = _swish(sd_ep + dot(e1, ep1[32:48, :]))
    hp = _swish(dot(hp, ep2[...]))
    hp = _swish(dot(hp, ep3[...]))
    hg = _swish(sd_eg + dot(e1, eg1[32:48, :]))
    hg = _swish(dot(hg, eg2[...]))
    hg = _sigmoid(dot(hg, eg3[...]))
    e2 = e1 + hp * hg * ew[...]

    sd_np, sd_ng = sd(np1[...]), sd(ng1[...])
    qp = _swish(sd_np + dot(e2, np1[32:48, :]))
    qp = _swish(dot(qp, np2[...]))
    qp = _swish(dot(qp, np3[...]))
    qg = _swish(sd_ng + dot(e2, ng1[32:48, :]))
    qg = _swish(dot(qg, ng2[...]))
    qg = _sigmoid(dot(qg, ng3[...]))
    ef_out[...] = e2
    msg_out[...] = qp * qg * nw[...]


def _conv_call(gsrc, gdst, s, ef, ew, nw, wbp, wbg, gc):
    e = gsrc.shape[0]
    grid = e // TE
    eb = lambda w: pl.BlockSpec((TE, w), lambda i: (i, 0))
    wb = lambda a: pl.BlockSpec(a.shape, lambda i: (0,) * a.ndim)
    ws = [wbp, wbg] + gc['edge_proj'] + gc['edge_gate'] + gc['node_proj'] + gc['node_gate']
    return pl.pallas_call(
        _conv_body,
        grid=(grid,),
        in_specs=[eb(32), eb(32), eb(16), eb(16), eb(16), eb(16)]
                 + [wb(w) for w in ws],
        out_specs=[eb(16), eb(16)],
        out_shape=[jax.ShapeDtypeStruct((e, 16), jnp.float32)] * 2,
        compiler_params=pltpu.CompilerParams(
            dimension_semantics=("parallel",)),
    )(gsrc, gdst, s, ef, ew, nw, *ws)


def _readout_body(tprev, p0, p1, nt, elem_ref, w1, b1, w2, b2, w3, b3,
                  out, acc, offacc):
    i = pl.program_id(0)
    n = tprev.shape[0]

    @pl.when(i == 0)
    def _init():
        acc[...] = jnp.zeros_like(acc)
        offacc[...] = jnp.zeros_like(offacc)

    nf = tprev[:, 0:NNF] + p0[...] + p1[...]
    acc[...] += jnp.sum(nf, axis=0, keepdims=True)
    oh = (nt[...] == lax.broadcasted_iota(jnp.int32, (n, N_ELEM), 1)).astype(jnp.float32)
    offacc[...] = offacc[...] + jnp.sum(oh * elem_ref[...])

    @pl.when(i == pl.num_programs(0) - 1)
    def _final():
        dot = lambda a, b: jnp.dot(a, b, preferred_element_type=jnp.float32)
        nv = acc[...] / float(N_TOTAL_NODES)
        h = _swish(dot(nv, w1[...]) + b1[...])
        h = _swish(dot(h, w2[...]) + b2[...])
        out[...] = dot(h, w3[...]) + b3[...] + offacc[...]


N_TOTAL_NODES = 50000


def _readout_call(tprev, p0, p1, node_type, elem_ref, fin):
    n = tprev.shape[0]
    global N_TOTAL_NODES
    N_TOTAL_NODES = n
    grid = n // TN
    wb = lambda a: pl.BlockSpec(a.shape, lambda i: (0,) * a.ndim)
    nb = lambda w: pl.BlockSpec((TN, w), lambda i: (i, 0))
    ws = [elem_ref.reshape(1, N_ELEM), fin['W1'], fin['b1'].reshape(1, -1),
          fin['W2'], fin['b2'].reshape(1, -1), fin['W3'], fin['b3'].reshape(1, 1)]
    return pl.pallas_call(
        _readout_body,
        grid=(grid,),
        in_specs=[nb(32), nb(16), nb(16), pl.BlockSpec((TN, 1), lambda i: (i, 0))]
                 + [wb(w) for w in ws],
        out_specs=pl.BlockSpec((1, 1), lambda i: (0, 0)),
        out_shape=jax.ShapeDtypeStruct((1, 1), jnp.float32),
        scratch_shapes=[pltpu.VMEM((1, 16), jnp.float32),
                        pltpu.VMEM((1, 1), jnp.float32)],
    )(tprev, p0, p1, node_type, *ws)


def _gather_rows(table, idx):
    return jnp.take(table, idx, axis=0)


def _scatter_add_rows(n_rows, idx, vals):
    acc = jnp.zeros((2, n_rows, vals.shape[1]), jnp.float32)
    acc = acc.at[0, idx].add(vals)
    return acc[0], acc[1]


def kernel(pos, state_attr, params, node_type, edge_index, lg_edge_index):
    n = pos.shape[0]
    e = edge_index.shape[1]
    src, dst = edge_index[0], edge_index[1]
    a, b = lg_edge_index[0], lg_edge_index[1]
    gc = params['conv']

    pos4 = jnp.pad(pos, ((0, 0), (0, 1)))
    psrc = _gather_rows(pos4, src)
    pdst = _gather_rows(pos4, dst)

    bee = params['b_edge_emb'].reshape(1, NEF)
    g_tab, ef0, ew, nw = _geom_call(psrc, pdst, params['W_edge_emb'], bee,
                                    gc['W_edge_w'], gc['W_node_w'])

    ga = _gather_rows(g_tab, a)
    gb = _gather_rows(g_tab, b)
    basisw = _triple_call(ga, gb)
    s0, s1 = _scatter_add_rows(e, b, basisw)
    s_tab = s0 + s1

    pad_w = lambda w: jnp.pad(w, ((0, 16 - DEGREE), (0, 0)))
    pad_b = lambda v: jnp.pad(v, (0, 16 - DEGREE)).reshape(1, 16)

    tb0 = params['three_body'][0]
    table = _prep0_call(node_type.reshape(n, 1),
                        params['W_node_emb'], params['b_node_emb'].reshape(1, NNF),
                        tb0['W_atom'], tb0['b_atom'].reshape(1, DEGREE))

    ef = ef0
    for i in range(N_BLOCKS):
        gsrc = _gather_rows(table, src)
        gdst = _gather_rows(table, dst)
        tb = params['three_body'][i]
        wbp = jnp.pad(tb['W_bond_p'], ((0, 16 - DEGREE), (0, 0)))
        wbg = jnp.pad(tb['W_bond_g'], ((0, 16 - DEGREE), (0, 0)))
        ef, msg = _conv_call(gsrc, gdst, s_tab, ef, ew, nw, wbp, wbg, gc)
        p0, p1 = _scatter_add_rows(n, dst, msg)
        if i < N_BLOCKS - 1:
            tbn = params['three_body'][i + 1]
            table = _prep_call(table, p0, p1, tbn['W_atom'],
                               tbn['b_atom'].reshape(1, DEGREE))
        else:
            out = _readout_call(table, p0, p1, node_type.reshape(n, 1),
                                params['elem_ref'], params['final'])
    return out


# speedup vs baseline: 1.5495x; 1.5495x over previous
import functools
import math

import jax
import jax.numpy as jnp
from jax import lax
from jax.experimental import pallas as pl
from jax.experimental.pallas import tpu as pltpu
from jax.experimental.pallas import tpu_sc as plsc

SC_CORES = 2
SC_TILES = 16
SC_WORKERS = SC_CORES * SC_TILES

MAX_N = 3
MAX_L = 3
CUTOFF = 5.0
THREE_CUTOFF = 4.0
NNF = 16
NEF = 16
UNITS = 64
N_BLOCKS = 3
N_ELEM = 16
DEGREE = MAX_N * MAX_L

TE = 2000
TT = 2000
TN = 2000


def _sigmoid(x):
    return 1.0 / (1.0 + jnp.exp(-x))


def _swish(x):
    return x * _sigmoid(x)


def _smooth_sbf_cols(bd):
    pi = math.pi
    cols = []
    for n in range(MAX_N):
        x1 = bd * ((n + 1.0) * pi / CUTOFF)
        x2 = bd * ((n + 2.0) * pi / CUTOFF)
        c = ((-1.0) ** n) * math.sqrt(2.0) * pi / CUTOFF ** 1.5 \
            * (n + 1.0) * (n + 2.0) / math.sqrt((n + 1.0) ** 2 + (n + 2.0) ** 2)
        cols.append(c * (jnp.sin(x1) / x1 + jnp.sin(x2) / x2))
    dn = [1.0]
    for i in range(1, MAX_N):
        e_i = i ** 2 * (i + 2) ** 2 / (4.0 * (i + 1) ** 4 + 1.0)
        dn.append(1.0 - e_i / dn[-1])
    gn = [cols[0]]
    for i in range(1, MAX_N):
        e_i = i ** 2 * (i + 2) ** 2 / (4.0 * (i + 1) ** 4 + 1.0)
        gn.append((cols[i] + math.sqrt(e_i / dn[i - 1]) * gn[-1]) / math.sqrt(dn[i]))
    return jnp.concatenate(gn, axis=1)


def _poly_cutoff(r, rc):
    ratio = r / rc
    v = 1.0 - 6.0 * ratio ** 5 + 15.0 * ratio ** 4 - 10.0 * ratio ** 3
    return jnp.where(r <= rc, v, 0.0)


def _geom_body(psrc, pdst, wee, bee, wew, wnw, g_out, ef0, ew, nw):
    bv = pdst[:, 0:3] - psrc[:, 0:3]
    bd = jnp.sqrt(jnp.sum(bv * bv, axis=1, keepdims=True) + 1e-12)
    cut = _poly_cutoff(bd, THREE_CUTOFF)
    rbf = _smooth_sbf_cols(bd)
    zcols = jnp.zeros((psrc.shape[0], 11), jnp.float32)
    g_out[...] = jnp.concatenate([bv, bd, cut, zcols], axis=1)
    ef0[...] = _swish(jnp.dot(rbf, wee[...], preferred_element_type=jnp.float32)
                      + bee[...])
    ew[...] = jnp.dot(rbf, wew[...], preferred_element_type=jnp.float32)
    nw[...] = jnp.dot(rbf, wnw[...], preferred_element_type=jnp.float32)


def _geom_call(psrc, pdst, wee, bee, wew, wnw):
    e = psrc.shape[0]
    grid = e // TE
    eb = lambda w: pl.BlockSpec((TE, w), lambda i: (i, 0))
    wb = lambda a: pl.BlockSpec(a.shape, lambda i: (0,) * a.ndim)
    return pl.pallas_call(
        _geom_body,
        grid=(grid,),
        in_specs=[eb(16), eb(16), wb(wee), wb(bee), wb(wew), wb(wnw)],
        out_specs=[eb(16), eb(16), eb(16), eb(16)],
        out_shape=[jax.ShapeDtypeStruct((e, 16), jnp.float32)] * 4,
        compiler_params=pltpu.CompilerParams(
            dimension_semantics=("parallel",)),
    )(psrc, pdst, wee, bee, wew, wnw)


def _triple_body(ga, gb, basisw):
    va = ga[:, 0:3]
    vb = gb[:, 0:3]
    na = ga[:, 3:4]
    nb = gb[:, 3:4]
    cos_t = jnp.clip(jnp.sum(va * vb, axis=1, keepdims=True) / (na * nb),
                     -1.0, 1.0)
    sbf = _smooth_sbf_cols(nb)
    p0 = jnp.ones_like(cos_t)
    p1 = cos_t
    p2 = (3.0 * cos_t * cos_t - 1.0) * 0.5
    leg = jnp.concatenate([p0, p1, p2], axis=1)
    w = ga[:, 4:5] * gb[:, 4:5]
    outer = [sbf[:, i:i + 1] * leg for i in range(MAX_N)]
    z = jnp.zeros((ga.shape[0], 16 - DEGREE), jnp.float32)
    basisw[...] = jnp.concatenate(outer + [z], axis=1) * w


def _triple_call(ga, gb):
    t = ga.shape[0]
    grid = t // TT
    tb = lambda w: pl.BlockSpec((TT, w), lambda i: (i, 0))
    return pl.pallas_call(
        _triple_body,
        grid=(grid,),
        in_specs=[tb(16), tb(16)],
        out_specs=tb(16),
        out_shape=jax.ShapeDtypeStruct((t, 16), jnp.float32),
        compiler_params=pltpu.CompilerParams(
            dimension_semantics=("parallel",)),
    )(ga, gb)


def _prep0_body(nt, wemb, bemb, watom, batom, table):
    n = nt.shape[0]
    oh = (nt[...] == lax.broadcasted_iota(jnp.int32, (n, N_ELEM), 1)).astype(jnp.float32)
    nf = _swish(jnp.dot(oh, wemb[...], preferred_element_type=jnp.float32)
                + bemb[...])
    atoms = _sigmoid(jnp.dot(nf, watom[...], preferred_element_type=jnp.float32)
                     + batom[...])
    z = jnp.zeros((n, 32 - NNF - DEGREE), jnp.float32)
    table[...] = jnp.concatenate([nf, atoms, z], axis=1)


def _prep0_call(node_type, wemb, bemb, watom, batom):
    n = node_type.shape[0]
    grid = n // TN
    wb = lambda a: pl.BlockSpec(a.shape, lambda i: (0,) * a.ndim)
    return pl.pallas_call(
        _prep0_body,
        grid=(grid,),
        in_specs=[pl.BlockSpec((TN, 1), lambda i: (i, 0)),
                  wb(wemb), wb(bemb), wb(watom), wb(batom)],
        out_specs=pl.BlockSpec((TN, 32), lambda i: (i, 0)),
        out_shape=jax.ShapeDtypeStruct((n, 32), jnp.float32),
        compiler_params=pltpu.CompilerParams(
            dimension_semantics=("parallel",)),
    )(node_type, wemb, bemb, watom, batom)


def _prep_body(tprev, p0, p1, watom, batom, table):
    n = tprev.shape[0]
    nf = tprev[:, 0:NNF] + p0[...] + p1[...]
    atoms = _sigmoid(jnp.dot(nf, watom[...], preferred_element_type=jnp.float32)
                     + batom[...])
    z = jnp.zeros((n, 32 - NNF - DEGREE), jnp.float32)
    table[...] = jnp.concatenate([nf, atoms, z], axis=1)


def _prep_call(tprev, p0, p1, watom, batom):
    n = tprev.shape[0]
    grid = n // TN
    wb = lambda a: pl.BlockSpec(a.shape, lambda i: (0,) * a.ndim)
    nb = lambda w: pl.BlockSpec((TN, w), lambda i: (i, 0))
    return pl.pallas_call(
        _prep_body,
        grid=(grid,),
        in_specs=[nb(32), nb(16), nb(16), wb(watom), wb(batom)],
        out_specs=nb(32),
        out_shape=jax.ShapeDtypeStruct((n, 32), jnp.float32),
        compiler_params=pltpu.CompilerParams(
            dimension_semantics=("parallel",)),
    )(tprev, p0, p1, watom, batom)


def _conv_body(gsrc, gdst, s, ef, ew, nw, wbp, wbg,
               ep1, ep2, ep3, eg1, eg2, eg3, np1, np2, np3, ng1, ng2, ng3,
               ef_out, msg_out):
    f32 = jnp.float32
    dot = lambda a, b: jnp.dot(a, b, preferred_element_type=f32)
    src16 = gsrc[:, 0:16]
    dst16 = gdst[:, 0:16]
    atoms = gdst[:, 16:32]
    nb = s[...] * atoms
    bond = _swish(dot(nb, wbp[...])) * _sigmoid(dot(nb, wbg[...]))
    e1 = ef[...] + bond

    def sd(w1):
        return dot(src16, w1[0:16, :]) + dot(dst16, w1[16:32, :])

    sd_ep, sd_eg = sd(ep1[...]), sd(eg1[...])
    hp = _swish(sd_ep + dot(e1, ep1[32:48, :]))
    hp = _swish(dot(hp, ep2[...]))
    hp = _swish(dot(hp, ep3[...]))
    hg = _swish(sd_eg + dot(e1, eg1[32:48, :]))
    hg = _swish(dot(hg, eg2[...]))
    hg = _sigmoid(dot(hg, eg3[...]))
    e2 = e1 + hp * hg * ew[...]

    sd_np, sd_ng = sd(np1[...]), sd(ng1[...])
    qp = _swish(sd_np + dot(e2, np1[32:48, :]))
    qp = _swish(dot(qp, np2[...]))
    qp = _swish(dot(qp, np3[...]))
    qg = _swish(sd_ng + dot(e2, ng1[32:48, :]))
    qg = _swish(dot(qg, ng2[...]))
    qg = _sigmoid(dot(qg, ng3[...]))
    ef_out[...] = e2
    msg_out[...] = qp * qg * nw[...]


def _conv_call(gsrc, gdst, s, ef, ew, nw, wbp, wbg, gc):
    e = gsrc.shape[0]
    grid = e // TE
    eb = lambda w: pl.BlockSpec((TE, w), lambda i: (i, 0))
    wb = lambda a: pl.BlockSpec(a.shape, lambda i: (0,) * a.ndim)
    ws = [wbp, wbg] + gc['edge_proj'] + gc['edge_gate'] + gc['node_proj'] + gc['node_gate']
    return pl.pallas_call(
        _conv_body,
        grid=(grid,),
        in_specs=[eb(32), eb(32), eb(16), eb(16), eb(16), eb(16)]
                 + [wb(w) for w in ws],
        out_specs=[eb(16), eb(16)],
        out_shape=[jax.ShapeDtypeStruct((e, 16), jnp.float32)] * 2,
        compiler_params=pltpu.CompilerParams(
            dimension_semantics=("parallel",)),
    )(gsrc, gdst, s, ef, ew, nw, *ws)


def _readout_body(tprev, p0, p1, nt, elem_ref, w1, b1, w2, b2, w3, b3,
                  out, acc, offacc):
    i = pl.program_id(0)
    n = tprev.shape[0]

    @pl.when(i == 0)
    def _init():
        acc[...] = jnp.zeros_like(acc)
        offacc[...] = jnp.zeros_like(offacc)

    nf = tprev[:, 0:NNF] + p0[...] + p1[...]
    acc[...] += jnp.sum(nf, axis=0, keepdims=True)
    oh = (nt[...] == lax.broadcasted_iota(jnp.int32, (n, N_ELEM), 1)).astype(jnp.float32)
    offacc[...] = offacc[...] + jnp.sum(oh * elem_ref[...])

    @pl.when(i == pl.num_programs(0) - 1)
    def _final():
        dot = lambda a, b: jnp.dot(a, b, preferred_element_type=jnp.float32)
        nv = acc[...] / float(N_TOTAL_NODES)
        h = _swish(dot(nv, w1[...]) + b1[...])
        h = _swish(dot(h, w2[...]) + b2[...])
        out[...] = dot(h, w3[...]) + b3[...] + offacc[...]


N_TOTAL_NODES = 50000


def _readout_call(tprev, p0, p1, node_type, elem_ref, fin):
    n = tprev.shape[0]
    global N_TOTAL_NODES
    N_TOTAL_NODES = n
    grid = n // TN
    wb = lambda a: pl.BlockSpec(a.shape, lambda i: (0,) * a.ndim)
    nb = lambda w: pl.BlockSpec((TN, w), lambda i: (i, 0))
    ws = [elem_ref.reshape(1, N_ELEM), fin['W1'], fin['b1'].reshape(1, -1),
          fin['W2'], fin['b2'].reshape(1, -1), fin['W3'], fin['b3'].reshape(1, 1)]
    return pl.pallas_call(
        _readout_body,
        grid=(grid,),
        in_specs=[nb(32), nb(16), nb(16), pl.BlockSpec((TN, 1), lambda i: (i, 0))]
                 + [wb(w) for w in ws],
        out_specs=pl.BlockSpec((1, 1), lambda i: (0, 0)),
        out_shape=jax.ShapeDtypeStruct((1, 1), jnp.float32),
        scratch_shapes=[pltpu.VMEM((1, 16), jnp.float32),
                        pltpu.VMEM((1, 1), jnp.float32)],
    )(tprev, p0, p1, node_type, *ws)


def _sc_gather2(table, idx0, idx1):
    k = idx0.shape[0]
    d = table.shape[1]
    kpw = k // SC_WORKERS
    ch = 1000
    n_ch = kpw // ch
    mesh = plsc.VectorSubcoreMesh(core_axis_name="c", subcore_axis_name="s")

    @functools.partial(
        pl.kernel, mesh=mesh,
        compiler_params=pltpu.CompilerParams(use_tc_tiling_on_sc=False),
        out_type=[jax.ShapeDtypeStruct((k, d), jnp.float32)] * 2,
        scratch_types=[pltpu.VMEM((ch,), jnp.int32),
                       pltpu.VMEM((ch, d), jnp.float32),
                       pltpu.SemaphoreType.DMA],
    )
    def gk(table_h, i0_h, i1_h, o0_h, o1_h, idx_v, rows_v, sem):
        wid = lax.axis_index("c") * SC_TILES + lax.axis_index("s")
        base = wid * kpw

        @pl.loop(0, n_ch)
        def _chunk(c):
            off = base + c * ch
            for ih, oh in ((i0_h, o0_h), (i1_h, o1_h)):
                pltpu.sync_copy(ih.at[pl.ds(off, ch)], idx_v)
                pltpu.async_copy(table_h.at[idx_v], rows_v, sem).wait()
                pltpu.sync_copy(rows_v, oh.at[pl.ds(off, ch)])

    return gk(table, idx0, idx1)


def _sc_scatter_node(msg, dst, n):
    e = msg.shape[0]
    epw = e // SC_WORKERS
    ch = 1000
    n_ch = epw // ch
    npt = n // SC_TILES
    zeros = jnp.zeros((n, 16), jnp.float32)
    mesh = plsc.VectorSubcoreMesh(core_axis_name="c", subcore_axis_name="s")

    @functools.partial(
        pl.kernel, mesh=mesh,
        compiler_params=pltpu.CompilerParams(use_tc_tiling_on_sc=False),
        out_type=jax.ShapeDtypeStruct((SC_CORES, n, 16), jnp.float32),
        scratch_types=[pltpu.VMEM((ch,), jnp.int32),
                       pltpu.VMEM((ch, 16), jnp.float32),
                       pltpu.VMEM_SHARED((n, 16), jnp.float32)],
    )
    def sk(msg_h, dst_h, zero_h, out_h, idx_v, rows_v, acc_sh):
        cid = lax.axis_index("c")
        sid = lax.axis_index("s")
        pltpu.sync_copy(zero_h.at[pl.ds(sid * npt, npt)],
                        acc_sh.at[pl.ds(sid * npt, npt)])
        plsc.subcore_barrier()
        base = (cid * SC_TILES + sid) * epw

        @pl.loop(0, n_ch)
        def _chunk(c):
            off = base + c * ch
            pltpu.sync_copy(dst_h.at[pl.ds(off, ch)], idx_v)
            pltpu.sync_copy(msg_h.at[pl.ds(off, ch)], rows_v)
            pltpu.sync_copy(rows_v, acc_sh.at[idx_v], add=True)

        plsc.subcore_barrier()
        pltpu.sync_copy(acc_sh.at[pl.ds(sid * npt, npt)],
                        out_h.at[cid, pl.ds(sid * npt, npt)])

    out = sk(msg, dst, zeros)
    return out[0], out[1]


def _sc_scatter_s(vals, b, e):
    t = vals.shape[0]
    chs = 80000
    n_chunks = e // chs
    cpc = n_chunks // SC_CORES
    tpt = t // SC_TILES
    ch2 = 2000
    n_ch2 = tpt // ch2
    zpt = (chs + SC_TILES) // SC_TILES
    wpt = chs // SC_TILES
    dummy = jnp.int32(chs)
    zeros = jnp.zeros((chs + SC_TILES, 16), jnp.float32)
    mesh = plsc.VectorSubcoreMesh(core_axis_name="c", subcore_axis_name="s")

    @functools.partial(
        pl.kernel, mesh=mesh,
        compiler_params=pltpu.CompilerParams(use_tc_tiling_on_sc=False),
        out_type=jax.ShapeDtypeStruct((e, 16), jnp.float32),
        scratch_types=[pltpu.VMEM((ch2,), jnp.int32),
                       pltpu.VMEM((ch2,), jnp.int32),
                       pltpu.VMEM((ch2, 16), jnp.float32),
                       pltpu.VMEM_SHARED((chs + SC_TILES, 16), jnp.float32)],
    )
    def sk(vals_h, b_h, zero_h, out_h, idx_v, idx2_v, rows_v, acc_sh):
        cid = lax.axis_index("c")
        sid = lax.axis_index("s")
        tb = sid * tpt
        for ci in range(cpc):
            chunk = ci * SC_CORES + cid
            lo = chunk * chs
            pltpu.sync_copy(zero_h.at[pl.ds(sid * zpt, zpt)],
                            acc_sh.at[pl.ds(sid * zpt, zpt)])
            plsc.subcore_barrier()

            @pl.loop(0, n_ch2)
            def _chunk(c2):
                off = tb + c2 * ch2
                pltpu.sync_copy(b_h.at[pl.ds(off, ch2)], idx_v)
                pltpu.sync_copy(vals_h.at[pl.ds(off, ch2)], rows_v)

                @pl.loop(0, ch2 // 16)
                def _vreg(j):
                    v = idx_v[pl.ds(j * 16, 16)]
                    lv = v - lo
                    m = (v >= lo) & (v < lo + chs)
                    idx2_v[pl.ds(j * 16, 16)] = jnp.where(m, lv, dummy)

                pltpu.sync_copy(rows_v, acc_sh.at[idx2_v], add=True)

            plsc.subcore_barrier()
            pltpu.sync_copy(acc_sh.at[pl.ds(sid * wpt, wpt)],
                            out_h.at[pl.ds(lo + sid * wpt, wpt)])
            plsc.subcore_barrier()

    return sk(vals, b, zeros)


def kernel(pos, state_attr, params, node_type, edge_index, lg_edge_index):
    n = pos.shape[0]
    e = edge_index.shape[1]
    src, dst = edge_index[0], edge_index[1]
    a, b = lg_edge_index[0], lg_edge_index[1]
    gc = params['conv']

    pos16 = jnp.pad(pos, ((0, 0), (0, 13)))
    psrc, pdst = _sc_gather2(pos16, src, dst)

    bee = params['b_edge_emb'].reshape(1, NEF)
    g_tab, ef0, ew, nw = _geom_call(psrc, pdst, params['W_edge_emb'], bee,
                                    gc['W_edge_w'], gc['W_node_w'])

    ga, gb = _sc_gather2(g_tab, a, b)
    basisw = _triple_call(ga, gb)
    s_tab = _sc_scatter_s(basisw, b, e)

    pad_w = lambda w: jnp.pad(w, ((0, 16 - DEGREE), (0, 0)))
    pad_b = lambda v: jnp.pad(v, (0, 16 - DEGREE)).reshape(1, 16)

    tb0 = params['three_body'][0]
    table = _prep0_call(node_type.reshape(n, 1),
                        params['W_node_emb'], params['b_node_emb'].reshape(1, NNF),
                        tb0['W_atom'], tb0['b_atom'].reshape(1, DEGREE))

    ef = ef0
    for i in range(N_BLOCKS):
        gsrc, gdst = _sc_gather2(table, src, dst)
        tb = params['three_body'][i]
        wbp = jnp.pad(tb['W_bond_p'], ((0, 16 - DEGREE), (0, 0)))
        wbg = jnp.pad(tb['W_bond_g'], ((0, 16 - DEGREE), (0, 0)))
        ef, msg = _conv_call(gsrc, gdst, s_tab, ef, ew, nw, wbp, wbg, gc)
        p0, p1 = _sc_scatter_node(msg, dst, n)
        if i < N_BLOCKS - 1:
            tbn = params['three_body'][i + 1]
            table = _prep_call(table, p0, p1, tbn['W_atom'],
                               tbn['b_atom'].reshape(1, DEGREE))
        else:
            out = _readout_call(table, p0, p1, node_type.reshape(n, 1),
                                params['elem_ref'], params['final'])
    return out


# speedup vs baseline: 2.5720x; 1.6599x over previous
import functools
import math

import jax
import jax.numpy as jnp
from jax import lax
from jax.experimental import pallas as pl
from jax.experimental.pallas import tpu as pltpu
from jax.experimental.pallas import tpu_sc as plsc

SC_CORES = 2
SC_TILES = 16
SC_WORKERS = SC_CORES * SC_TILES

MAX_N = 3
MAX_L = 3
CUTOFF = 5.0
THREE_CUTOFF = 4.0
NNF = 16
NEF = 16
UNITS = 64
N_BLOCKS = 3
N_ELEM = 16
DEGREE = MAX_N * MAX_L

TE = 4000
TT = 8000
TN = 10000


def _sigmoid(x):
    return 1.0 / (1.0 + jnp.exp(-x))


def _swish(x):
    return x * _sigmoid(x)


def _sbf_mix_matrix():
    pi = math.pi
    m = [[0.0] * MAX_N for _ in range(MAX_N + 1)]
    for n in range(MAX_N):
        c = ((-1.0) ** n) * math.sqrt(2.0) * pi / CUTOFF ** 1.5 \
            * (n + 1.0) * (n + 2.0) / math.sqrt((n + 1.0) ** 2 + (n + 2.0) ** 2)
        m[n][n] += c
        m[n + 1][n] += c
    dn = [1.0]
    for i in range(1, MAX_N):
        e_i = i ** 2 * (i + 2) ** 2 / (4.0 * (i + 1) ** 4 + 1.0)
        dn.append(1.0 - e_i / dn[-1])
    lmat = [[0.0] * MAX_N for _ in range(MAX_N)]
    lmat[0][0] = 1.0
    for i in range(1, MAX_N):
        e_i = i ** 2 * (i + 2) ** 2 / (4.0 * (i + 1) ** 4 + 1.0)
        s_i = math.sqrt(e_i / dn[i - 1])
        t_i = math.sqrt(dn[i])
        for r in range(MAX_N):
            lmat[r][i] = (((1.0 if r == i else 0.0) + s_i * lmat[r][i - 1]) / t_i
                          if r <= i else 0.0)
    c = [[sum(m[k][r] * lmat[r][n] for r in range(MAX_N)) for n in range(MAX_N)]
         for k in range(MAX_N + 1)]
    return jnp.asarray(c, jnp.float32)


_SBF_C = _sbf_mix_matrix()
_SBF_K = jnp.asarray([[(k + 1) * math.pi / CUTOFF for k in range(MAX_N + 1)]],
                     jnp.float32)


def _poly_cutoff(r, rc):
    ratio = r / rc
    v = 1.0 - 6.0 * ratio ** 5 + 15.0 * ratio ** 4 - 10.0 * ratio ** 3
    return jnp.where(r <= rc, v, 0.0)


def _geom_body(psrc, pdst, kvec, cmix, cee, bee, cew, cnw, g_out, ef0, ew, nw):
    dot = lambda a, b: jnp.dot(a, b, preferred_element_type=jnp.float32)
    bv = pdst[:, 0:4] - psrc[:, 0:4]
    sq = bv * bv
    bd = jnp.sqrt(sq[:, 0:1] + sq[:, 1:2] + sq[:, 2:3] + 1e-12)
    cut = _poly_cutoff(bd, THREE_CUTOFF)
    xk = bd * kvec[...]
    inv = jnp.sin(xk) / xk
    rbf = dot(inv, cmix[...])
    zcols = jnp.zeros((psrc.shape[0], 8), jnp.float32)
    g_out[...] = jnp.concatenate([bv[:, 0:3], bd, cut, rbf, zcols], axis=1)
    ef0[...] = _swish(dot(inv, cee[...]) + bee[...])
    ew[...] = dot(inv, cew[...])
    nw[...] = dot(inv, cnw[...])


def _geom_call(psrc, pdst, wee, bee, wew, wnw):
    e = psrc.shape[0]
    grid = e // TE
    eb = lambda w: pl.BlockSpec((TE, w), lambda i: (i, 0))
    wb = lambda a: pl.BlockSpec(a.shape, lambda i: (0,) * a.ndim)
    cee = _SBF_C @ wee
    cew = _SBF_C @ wew
    cnw = _SBF_C @ wnw
    ws = [_SBF_K, _SBF_C, cee, bee, cew, cnw]
    return pl.pallas_call(
        _geom_body,
        grid=(grid,),
        in_specs=[eb(16), eb(16)] + [wb(w) for w in ws],
        out_specs=[eb(16), eb(16), eb(16), eb(16)],
        out_shape=[jax.ShapeDtypeStruct((e, 16), jnp.float32)] * 4,
        compiler_params=pltpu.CompilerParams(
            dimension_semantics=("parallel",)),
    )(psrc, pdst, *ws)


def _triple_body(ga, gb, basisw):
    va = ga[:, 0:3]
    vb = gb[:, 0:3]
    na = ga[:, 3:4]
    nb = gb[:, 3:4]
    cos_t = jnp.clip(jnp.sum(va * vb, axis=1, keepdims=True) / (na * nb),
                     -1.0, 1.0)
    sbf = gb[:, 5:8]
    p0 = jnp.ones_like(cos_t)
    p1 = cos_t
    p2 = (3.0 * cos_t * cos_t - 1.0) * 0.5
    leg = jnp.concatenate([p0, p1, p2], axis=1)
    w = ga[:, 4:5] * gb[:, 4:5]
    outer = [sbf[:, i:i + 1] * leg for i in range(MAX_N)]
    z = jnp.zeros((ga.shape[0], 16 - DEGREE), jnp.float32)
    basisw[...] = jnp.concatenate(outer + [z], axis=1) * w


def _triple_call(ga, gb):
    t = ga.shape[0]
    grid = t // TT
    tb = lambda w: pl.BlockSpec((TT, w), lambda i: (i, 0))
    return pl.pallas_call(
        _triple_body,
        grid=(grid,),
        in_specs=[tb(16), tb(16)],
        out_specs=tb(16),
        out_shape=jax.ShapeDtypeStruct((t, 16), jnp.float32),
        compiler_params=pltpu.CompilerParams(
            dimension_semantics=("parallel",)),
    )(ga, gb)


def _prep0_body(nt, wemb, bemb, watom, batom, table):
    n = nt.shape[0]
    oh = (nt[...] == lax.broadcasted_iota(jnp.int32, (n, N_ELEM), 1)).astype(jnp.float32)
    nf = _swish(jnp.dot(oh, wemb[...], preferred_element_type=jnp.float32)
                + bemb[...])
    atoms = _sigmoid(jnp.dot(nf, watom[...], preferred_element_type=jnp.float32)
                     + batom[...])
    z = jnp.zeros((n, 32 - NNF - DEGREE), jnp.float32)
    table[...] = jnp.concatenate([nf, atoms, z], axis=1)


def _prep0_call(node_type, wemb, bemb, watom, batom):
    n = node_type.shape[0]
    grid = n // TN
    wb = lambda a: pl.BlockSpec(a.shape, lambda i: (0,) * a.ndim)
    return pl.pallas_call(
        _prep0_body,
        grid=(grid,),
        in_specs=[pl.BlockSpec((TN, 1), lambda i: (i, 0)),
                  wb(wemb), wb(bemb), wb(watom), wb(batom)],
        out_specs=pl.BlockSpec((TN, 32), lambda i: (i, 0)),
        out_shape=jax.ShapeDtypeStruct((n, 32), jnp.float32),
        compiler_params=pltpu.CompilerParams(
            dimension_semantics=("parallel",)),
    )(node_type, wemb, bemb, watom, batom)


def _prep_body(tprev, p0, p1, watom, batom, table):
    n = tprev.shape[0]
    nf = tprev[:, 0:NNF] + p0[...] + p1[...]
    atoms = _sigmoid(jnp.dot(nf, watom[...], preferred_element_type=jnp.float32)
                     + batom[...])
    z = jnp.zeros((n, 32 - NNF - DEGREE), jnp.float32)
    table[...] = jnp.concatenate([nf, atoms, z], axis=1)


def _prep_call(tprev, p0, p1, watom, batom):
    n = tprev.shape[0]
    grid = n // TN
    wb = lambda a: pl.BlockSpec(a.shape, lambda i: (0,) * a.ndim)
    nb = lambda w: pl.BlockSpec((TN, w), lambda i: (i, 0))
    return pl.pallas_call(
        _prep_body,
        grid=(grid,),
        in_specs=[nb(32), nb(16), nb(16), wb(watom), wb(batom)],
        out_specs=nb(32),
        out_shape=jax.ShapeDtypeStruct((n, 32), jnp.float32),
        compiler_params=pltpu.CompilerParams(
            dimension_semantics=("parallel",)),
    )(tprev, p0, p1, watom, batom)


def _conv_body(gsrc, gdst, s, ef, ew, nw, wbond, wsd, we_e, we_n,
               ep2, ep3, eg2, eg3, np2, np3, ng2, ng3,
               ef_out, msg_out):
    f32 = jnp.float32
    dot = lambda a, b: jnp.dot(a, b, preferred_element_type=f32)
    src16 = gsrc[:, 0:16]
    dst16 = gdst[:, 0:16]
    atoms = gdst[:, 16:32]
    nb = s[...] * atoms
    bnd = dot(nb, wbond[...])
    bond = _swish(bnd[:, 0:16]) * _sigmoid(bnd[:, 16:32])
    e1 = ef[...] + bond

    base = dot(src16, wsd[0:16, :]) + dot(dst16, wsd[16:32, :])
    e1c = dot(e1, we_e[...])
    hp = _swish(base[:, 0:UNITS] + e1c[:, 0:UNITS])
    hg = _swish(base[:, UNITS:2 * UNITS] + e1c[:, UNITS:2 * UNITS])
    hp = _swish(dot(hp, ep2[...]))
    hp = _swish(dot(hp, ep3[...]))
    hg = _swish(dot(hg, eg2[...]))
    hg = _sigmoid(dot(hg, eg3[...]))
    e2 = e1 + hp * hg * ew[...]

    e2c = dot(e2, we_n[...])
    qp = _swish(base[:, 2 * UNITS:3 * UNITS] + e2c[:, 0:UNITS])
    qg = _swish(base[:, 3 * UNITS:4 * UNITS] + e2c[:, UNITS:2 * UNITS])
    qp = _swish(dot(qp, np2[...]))
    qp = _swish(dot(qp, np3[...]))
    qg = _swish(dot(qg, ng2[...]))
    qg = _sigmoid(dot(qg, ng3[...]))
    ef_out[...] = e2
    msg_out[...] = qp * qg * nw[...]


def _conv_call(gsrc, gdst, s, ef, ew, nw, wbp, wbg, gc):
    e = gsrc.shape[0]
    grid = e // TE
    eb = lambda w: pl.BlockSpec((TE, w), lambda i: (i, 0))
    wb = lambda a: pl.BlockSpec(a.shape, lambda i: (0,) * a.ndim)
    ep1, ep2, ep3 = gc['edge_proj']
    eg1, eg2, eg3 = gc['edge_gate']
    np1, np2, np3 = gc['node_proj']
    ng1, ng2, ng3 = gc['node_gate']
    wbond = jnp.concatenate([wbp, wbg], axis=1)
    wsd = jnp.concatenate([w[0:32, :] for w in (ep1, eg1, np1, ng1)], axis=1)
    we_e = jnp.concatenate([ep1[32:48, :], eg1[32:48, :]], axis=1)
    we_n = jnp.concatenate([np1[32:48, :], ng1[32:48, :]], axis=1)
    ws = [wbond, wsd, we_e, we_n, ep2, ep3, eg2, eg3, np2, np3, ng2, ng3]
    return pl.pallas_call(
        _conv_body,
        grid=(grid,),
        in_specs=[eb(32), eb(32), eb(16), eb(16), eb(16), eb(16)]
                 + [wb(w) for w in ws],
        out_specs=[eb(16), eb(16)],
        out_shape=[jax.ShapeDtypeStruct((e, 16), jnp.float32)] * 2,
        compiler_params=pltpu.CompilerParams(
            dimension_semantics=("parallel",)),
    )(gsrc, gdst, s, ef, ew, nw, *ws)


def _readout_body(tprev, p0, p1, nt, elem_ref, w1, b1, w2, b2, w3, b3,
                  out, acc, offacc):
    i = pl.program_id(0)
    n = tprev.shape[0]

    @pl.when(i == 0)
    def _init():
        acc[...] = jnp.zeros_like(acc)
        offacc[...] = jnp.zeros_like(offacc)

    nf = tprev[:, 0:NNF] + p0[...] + p1[...]
    acc[...] += jnp.sum(nf, axis=0, keepdims=True)
    oh = (nt[...] == lax.broadcasted_iota(jnp.int32, (n, N_ELEM), 1)).astype(jnp.float32)
    offacc[...] = offacc[...] + jnp.sum(oh * elem_ref[...])

    @pl.when(i == pl.num_programs(0) - 1)
    def _final():
        dot = lambda a, b: jnp.dot(a, b, preferred_element_type=jnp.float32)
        nv = acc[...] / float(N_TOTAL_NODES)
        h = _swish(dot(nv, w1[...]) + b1[...])
        h = _swish(dot(h, w2[...]) + b2[...])
        out[...] = dot(h, w3[...]) + b3[...] + offacc[...]


N_TOTAL_NODES = 50000


def _readout_call(tprev, p0, p1, node_type, elem_ref, fin):
    n = tprev.shape[0]
    global N_TOTAL_NODES
    N_TOTAL_NODES = n
    grid = n // TN
    wb = lambda a: pl.BlockSpec(a.shape, lambda i: (0,) * a.ndim)
    nb = lambda w: pl.BlockSpec((TN, w), lambda i: (i, 0))
    ws = [elem_ref.reshape(1, N_ELEM), fin['W1'], fin['b1'].reshape(1, -1),
          fin['W2'], fin['b2'].reshape(1, -1), fin['W3'], fin['b3'].reshape(1, 1)]
    return pl.pallas_call(
        _readout_body,
        grid=(grid,),
        in_specs=[nb(32), nb(16), nb(16), pl.BlockSpec((TN, 1), lambda i: (i, 0))]
                 + [wb(w) for w in ws],
        out_specs=pl.BlockSpec((1, 1), lambda i: (0, 0)),
        out_shape=jax.ShapeDtypeStruct((1, 1), jnp.float32),
        scratch_shapes=[pltpu.VMEM((1, 16), jnp.float32),
                        pltpu.VMEM((1, 1), jnp.float32)],
    )(tprev, p0, p1, node_type, *ws)


def _sc_gather2(table, idx0, idx1):
    k = idx0.shape[0]
    d = table.shape[1]
    kpw = k // SC_WORKERS
    ch = 1000
    n_ch = kpw // ch
    mesh = plsc.VectorSubcoreMesh(core_axis_name="c", subcore_axis_name="s")

    @functools.partial(
        pl.kernel, mesh=mesh,
        compiler_params=pltpu.CompilerParams(use_tc_tiling_on_sc=False),
        out_type=[jax.ShapeDtypeStruct((k, d), jnp.float32)] * 2,
        scratch_types=[pltpu.VMEM((ch,), jnp.int32),
                       pltpu.VMEM((ch, d), jnp.float32),
                       pltpu.SemaphoreType.DMA],
    )
    def gk(table_h, i0_h, i1_h, o0_h, o1_h, idx_v, rows_v, sem):
        wid = lax.axis_index("c") * SC_TILES + lax.axis_index("s")
        base = wid * kpw

        @pl.loop(0, n_ch)
        def _chunk(c):
            off = base + c * ch
            for ih, oh in ((i0_h, o0_h), (i1_h, o1_h)):
                pltpu.sync_copy(ih.at[pl.ds(off, ch)], idx_v)
                pltpu.async_copy(table_h.at[idx_v], rows_v, sem).wait()
                pltpu.sync_copy(rows_v, oh.at[pl.ds(off, ch)])

    return gk(table, idx0, idx1)


def _sc_scatter_node(msg, dst, n):
    e = msg.shape[0]
    epw = e // SC_WORKERS
    ch = 1000
    n_ch = epw // ch
    npt = n // SC_TILES
    zeros = jnp.zeros((n, 16), jnp.float32)
    mesh = plsc.VectorSubcoreMesh(core_axis_name="c", subcore_axis_name="s")

    @functools.partial(
        pl.kernel, mesh=mesh,
        compiler_params=pltpu.CompilerParams(use_tc_tiling_on_sc=False),
        out_type=jax.ShapeDtypeStruct((SC_CORES, n, 16), jnp.float32),
        scratch_types=[pltpu.VMEM((ch,), jnp.int32),
                       pltpu.VMEM((ch, 16), jnp.float32),
                       pltpu.VMEM_SHARED((n, 16), jnp.float32)],
    )
    def sk(msg_h, dst_h, zero_h, out_h, idx_v, rows_v, acc_sh):
        cid = lax.axis_index("c")
        sid = lax.axis_index("s")
        pltpu.sync_copy(zero_h.at[pl.ds(sid * npt, npt)],
                        acc_sh.at[pl.ds(sid * npt, npt)])
        plsc.subcore_barrier()
        base = (cid * SC_TILES + sid) * epw

        @pl.loop(0, n_ch)
        def _chunk(c):
            off = base + c * ch
            pltpu.sync_copy(dst_h.at[pl.ds(off, ch)], idx_v)
            pltpu.sync_copy(msg_h.at[pl.ds(off, ch)], rows_v)
            pltpu.sync_copy(rows_v, acc_sh.at[idx_v], add=True)

        plsc.subcore_barrier()
        pltpu.sync_copy(acc_sh.at[pl.ds(sid * npt, npt)],
                        out_h.at[cid, pl.ds(sid * npt, npt)])

    out = sk(msg, dst, zeros)
    return out[0], out[1]


def _sc_scatter_s(vals, b, e):
    t = vals.shape[0]
    chs = 80000
    n_chunks = e // chs
    cpc = n_chunks // SC_CORES
    tpt = t // SC_TILES
    ch2 = 2000
    n_ch2 = tpt // ch2
    zpt = (chs + SC_TILES) // SC_TILES
    wpt = chs // SC_TILES
    dummy = jnp.int32(chs)
    zeros = jnp.zeros((chs + SC_TILES, 16), jnp.float32)
    mesh = plsc.VectorSubcoreMesh(core_axis_name="c", subcore_axis_name="s")

    @functools.partial(
        pl.kernel, mesh=mesh,
        compiler_params=pltpu.CompilerParams(use_tc_tiling_on_sc=False),
        out_type=jax.ShapeDtypeStruct((e, 16), jnp.float32),
        scratch_types=[pltpu.VMEM((ch2,), jnp.int32),
                       pltpu.VMEM((ch2,), jnp.int32),
                       pltpu.VMEM((ch2, 16), jnp.float32),
                       pltpu.VMEM_SHARED((chs + SC_TILES, 16), jnp.float32)],
    )
    def sk(vals_h, b_h, zero_h, out_h, idx_v, idx2_v, rows_v, acc_sh):
        cid = lax.axis_index("c")
        sid = lax.axis_index("s")
        tb = sid * tpt
        for ci in range(cpc):
            chunk = ci * SC_CORES + cid
            lo = chunk * chs
            pltpu.sync_copy(zero_h.at[pl.ds(sid * zpt, zpt)],
                            acc_sh.at[pl.ds(sid * zpt, zpt)])
            plsc.subcore_barrier()

            @pl.loop(0, n_ch2)
            def _chunk(c2):
                off = tb + c2 * ch2
                pltpu.sync_copy(b_h.at[pl.ds(off, ch2)], idx_v)
                pltpu.sync_copy(vals_h.at[pl.ds(off, ch2)], rows_v)

                @pl.loop(0, ch2 // 16)
                def _vreg(j):
                    v = idx_v[pl.ds(j * 16, 16)]
                    lv = v - lo
                    m = (v >= lo) & (v < lo + chs)
                    idx2_v[pl.ds(j * 16, 16)] = jnp.where(m, lv, dummy)

                pltpu.sync_copy(rows_v, acc_sh.at[idx2_v], add=True)

            plsc.subcore_barrier()
            pltpu.sync_copy(acc_sh.at[pl.ds(sid * wpt, wpt)],
                            out_h.at[pl.ds(lo + sid * wpt, wpt)])
            plsc.subcore_barrier()

    return sk(vals, b, zeros)


def kernel(pos, state_attr, params, node_type, edge_index, lg_edge_index):
    n = pos.shape[0]
    e = edge_index.shape[1]
    src, dst = edge_index[0], edge_index[1]
    a, b = lg_edge_index[0], lg_edge_index[1]
    gc = params['conv']

    pos16 = jnp.pad(pos, ((0, 0), (0, 13)))
    psrc, pdst = _sc_gather2(pos16, src, dst)

    bee = params['b_edge_emb'].reshape(1, NEF)
    g_tab, ef0, ew, nw = _geom_call(psrc, pdst, params['W_edge_emb'], bee,
                                    gc['W_edge_w'], gc['W_node_w'])

    ga, gb = _sc_gather2(g_tab, a, b)
    basisw = _triple_call(ga, gb)
    s_tab = _sc_scatter_s(basisw, b, e)

    pad_w = lambda w: jnp.pad(w, ((0, 16 - DEGREE), (0, 0)))
    pad_b = lambda v: jnp.pad(v, (0, 16 - DEGREE)).reshape(1, 16)

    tb0 = params['three_body'][0]
    table = _prep0_call(node_type.reshape(n, 1),
                        params['W_node_emb'], params['b_node_emb'].reshape(1, NNF),
                        tb0['W_atom'], tb0['b_atom'].reshape(1, DEGREE))

    ef = ef0
    for i in range(N_BLOCKS):
        gsrc, gdst = _sc_gather2(table, src, dst)
        tb = params['three_body'][i]
        wbp = jnp.pad(tb['W_bond_p'], ((0, 16 - DEGREE), (0, 0)))
        wbg = jnp.pad(tb['W_bond_g'], ((0, 16 - DEGREE), (0, 0)))
        ef, msg = _conv_call(gsrc, gdst, s_tab, ef, ew, nw, wbp, wbg, gc)
        p0, p1 = _sc_scatter_node(msg, dst, n)
        if i < N_BLOCKS - 1:
            tbn = params['three_body'][i + 1]
            table = _prep_call(table, p0, p1, tbn['W_atom'],
                               tbn['b_atom'].reshape(1, DEGREE))
        else:
            out = _readout_call(table, p0, p1, node_type.reshape(n, 1),
                                params['elem_ref'], params['final'])
    return out
